# Initial kernel scaffold; baseline (speedup 1.0000x reference)
import jax
import jax.numpy as jnp
from jax.experimental import pallas as pl


def kernel(x, ix, Wf, bf, Wg, bg, Wh, bh):
    raise NotImplementedError("write your pallas kernel here")


# speedup vs baseline: 2.2219x; 2.2219x over previous
import functools

import jax
import jax.numpy as jnp
from jax import lax
from jax.experimental import pallas as pl
from jax.experimental.pallas import tpu as pltpu
from jax.experimental.pallas import tpu_sc as plsc

N = 320000
D = 128
S = 10000
SZ = 10240
NC, NS = 2, 16
NW = NC * NS
PER_W = N // NW
CHUNK = 80
N_CH = PER_W // CHUNK
ZROWS = 128
RPT = SZ // NS
ZPT = SZ // NS

_mesh = plsc.VectorSubcoreMesh(core_axis_name="c", subcore_axis_name="s")


def _gates_body(x_ref, wg_ref, bg_ref, e_ref):
    g = jnp.dot(x_ref[...], wg_ref[...], preferred_element_type=jnp.float32)
    e_ref[...] = jnp.exp(g + bg_ref[...])


def _gates(x2, Wg, bg):
    B1 = 4000
    return pl.pallas_call(
        _gates_body,
        grid=(N // B1,),
        in_specs=[
            pl.BlockSpec((B1, D), lambda i: (i, 0)),
            pl.BlockSpec((D, 1), lambda i: (0, 0)),
            pl.BlockSpec((1, 1), lambda i: (0, 0)),
        ],
        out_specs=pl.BlockSpec((B1, 1), lambda i: (i, 0)),
        out_shape=jax.ShapeDtypeStruct((N, 1), jnp.float32),
    )(x2, Wg, bg.reshape(1, 1))


@functools.partial(
    pl.kernel,
    out_type=(
        jax.ShapeDtypeStruct((NC, SZ, D), jnp.float32),
        jax.ShapeDtypeStruct((NC, SZ), jnp.float32),
    ),
    mesh=_mesh,
    scratch_types=[
        pltpu.VMEM((CHUNK, D), jnp.float32),
        pltpu.VMEM((CHUNK,), jnp.float32),
        pltpu.VMEM((CHUNK,), jnp.int32),
        pltpu.VMEM((ZROWS, D), jnp.float32),
        pltpu.VMEM((ZPT,), jnp.float32),
        pltpu.VMEM_SHARED((SZ, D), jnp.float32),
        pltpu.VMEM_SHARED((SZ,), jnp.float32),
    ],
)
def _accumulate(x_hbm, e_hbm, ix_hbm, xbar_hbm, z_hbm,
                rows_v, e_v, ix_v, zero2_v, zero1_v, xbar_sh, z_sh):
    c = lax.axis_index("c")
    s = lax.axis_index("s")
    base = (c * NS + s) * PER_W

    zvec = jnp.zeros((16,), jnp.float32)

    def _zfill2(i, _):
        for v in range(D // 16):
            zero2_v[i, pl.ds(v * 16, 16)] = zvec
        return 0

    lax.fori_loop(0, ZROWS, _zfill2, 0)

    def _zfill1(i, _):
        zero1_v[pl.ds(i * 16, 16)] = zvec
        return 0

    lax.fori_loop(0, ZPT // 16, _zfill1, 0)

    for j in range(RPT // ZROWS):
        pltpu.sync_copy(zero2_v, xbar_sh.at[pl.ds(s * RPT + j * ZROWS, ZROWS)])
    pltpu.sync_copy(zero1_v, z_sh.at[pl.ds(s * ZPT, ZPT)])
    plsc.subcore_barrier()

    def _chunk(t, _):
        off = base + t * CHUNK
        pltpu.sync_copy(x_hbm.at[pl.ds(off, CHUNK)], rows_v)
        pltpu.sync_copy(e_hbm.at[pl.ds(off, CHUNK)], e_v)
        pltpu.sync_copy(ix_hbm.at[pl.ds(off, CHUNK)], ix_v)

        def _group(g, _):
            e16 = e_v[pl.ds(g * 16, 16)]
            for j in range(16):
                r = g * 16 + j
                splat = jnp.broadcast_to(e16[j], (16,))
                for v in range(D // 16):
                    sl = pl.ds(v * 16, 16)
                    rows_v[r, sl] = rows_v[r, sl] * splat
            return 0

        lax.fori_loop(0, CHUNK // 16, _group, 0)
        pltpu.sync_copy(rows_v, xbar_sh.at[ix_v], add=True)
        pltpu.sync_copy(e_v, z_sh.at[ix_v], add=True)
        return 0

    lax.fori_loop(0, N_CH, _chunk, 0)
    plsc.subcore_barrier()

    pltpu.sync_copy(xbar_sh.at[pl.ds(s * RPT, RPT)],
                    xbar_hbm.at[c, pl.ds(s * RPT, RPT)])
    pltpu.sync_copy(z_sh.at[pl.ds(s * ZPT, ZPT)],
                    z_hbm.at[c, pl.ds(s * ZPT, ZPT)])


def _combine_body(xbar_ref, z_ref, wf_ref, bf_ref, wh_ref, bh_ref, seg_ref):
    xbar = xbar_ref[0, :S] + xbar_ref[1, :S]
    z = z_ref[0, :S] + z_ref[1, :S]
    inv = jnp.where(z > 0, 1.0 / jnp.where(z > 0, z, 1.0), 0.0)
    ybar = xbar * inv[:, None]
    t = jnp.dot(ybar, wf_ref[...], preferred_element_type=jnp.float32)
    t = t + bf_ref[...]
    o = jnp.dot(t, wh_ref[...], preferred_element_type=jnp.float32)
    seg_ref[...] = o + bh_ref[...]


def _combine(xbar2, z2, Wf, bf, Wh, bh):
    return pl.pallas_call(
        _combine_body,
        out_shape=jax.ShapeDtypeStruct((S, D), jnp.float32),
    )(xbar2, z2, Wf, bf.reshape(1, D), Wh, bh.reshape(1, D))


@functools.partial(
    pl.kernel,
    out_type=jax.ShapeDtypeStruct((N, D), jnp.float32),
    mesh=_mesh,
    scratch_types=[
        pltpu.VMEM((CHUNK, D), jnp.float32),
        pltpu.VMEM((CHUNK,), jnp.int32),
        pltpu.SemaphoreType.DMA,
    ],
)
def _expand(seg_hbm, ix_hbm, out_hbm, rows_v, ix_v, sem):
    c = lax.axis_index("c")
    s = lax.axis_index("s")
    base = (c * NS + s) * PER_W

    def _chunk(t, _):
        off = base + t * CHUNK
        pltpu.sync_copy(ix_hbm.at[pl.ds(off, CHUNK)], ix_v)
        pltpu.async_copy(seg_hbm.at[ix_v], rows_v, sem).wait()
        pltpu.sync_copy(rows_v, out_hbm.at[pl.ds(off, CHUNK)])
        return 0

    lax.fori_loop(0, N_CH, _chunk, 0)


def kernel(x, ix, Wf, bf, Wg, bg, Wh, bh):
    x2 = x.reshape(N, D)
    ixi = ix.reshape(N).astype(jnp.int32)
    e = _gates(x2, Wg, bg).reshape(N)
    xbar2, z2 = _accumulate(x2, e, ixi)
    seg = _combine(xbar2, z2, Wf, bf, Wh, bh)
    out = _expand(seg, ixi)
    return out.reshape(1, N, D)


# speedup vs baseline: 3.7664x; 1.6952x over previous
import functools

import jax
import jax.numpy as jnp
from jax import lax
from jax.experimental import pallas as pl
from jax.experimental.pallas import tpu as pltpu
from jax.experimental.pallas import tpu_sc as plsc

N = 320000
D = 128
S = 10000
SZ = 10240
NC, NS = 2, 16
NW = NC * NS
PER_W = N // NW
CHUNK = 80
N_CH = PER_W // CHUNK
ZROWS = 32
RPT = SZ // NS
ZPT = SZ // NS

_mesh = plsc.VectorSubcoreMesh(core_axis_name="c", subcore_axis_name="s")


def _gates_body(x_ref, wg_ref, bg_ref, e_ref):
    g = jnp.dot(x_ref[...], wg_ref[...], preferred_element_type=jnp.float32)
    e_ref[...] = jnp.exp(g + bg_ref[...])


def _gates(x2, Wg, bg):
    B1 = 4000
    return pl.pallas_call(
        _gates_body,
        grid=(N // B1,),
        in_specs=[
            pl.BlockSpec((B1, D), lambda i: (i, 0)),
            pl.BlockSpec((D, 1), lambda i: (0, 0)),
            pl.BlockSpec((1, 1), lambda i: (0, 0)),
        ],
        out_specs=pl.BlockSpec((B1, 1), lambda i: (i, 0)),
        out_shape=jax.ShapeDtypeStruct((N, 1), jnp.float32),
    )(x2, Wg, bg.reshape(1, 1))


NB = 3


@functools.partial(
    pl.kernel,
    out_type=(
        jax.ShapeDtypeStruct((NC, SZ, D), jnp.float32),
        jax.ShapeDtypeStruct((NC, SZ), jnp.float32),
    ),
    mesh=_mesh,
    scratch_types=[
        pltpu.VMEM((CHUNK, D), jnp.float32),
        pltpu.VMEM((CHUNK, D), jnp.float32),
        pltpu.VMEM((CHUNK, D), jnp.float32),
        pltpu.VMEM((CHUNK,), jnp.float32),
        pltpu.VMEM((CHUNK,), jnp.float32),
        pltpu.VMEM((CHUNK,), jnp.float32),
        pltpu.VMEM((CHUNK,), jnp.int32),
        pltpu.VMEM((CHUNK,), jnp.int32),
        pltpu.VMEM((CHUNK,), jnp.int32),
        pltpu.VMEM((ZROWS, D), jnp.float32),
        pltpu.VMEM((ZPT,), jnp.float32),
        pltpu.VMEM_SHARED((SZ, D), jnp.float32),
        pltpu.VMEM_SHARED((SZ,), jnp.float32),
        pltpu.SemaphoreType.DMA,
        pltpu.SemaphoreType.DMA,
        pltpu.SemaphoreType.DMA,
        pltpu.SemaphoreType.DMA,
        pltpu.SemaphoreType.DMA,
        pltpu.SemaphoreType.DMA,
        pltpu.SemaphoreType.DMA,
        pltpu.SemaphoreType.DMA,
        pltpu.SemaphoreType.DMA,
    ],
)
def _accumulate(x_hbm, e_hbm, ix_hbm, xbar_hbm, z_hbm,
                rows0, rows1, rows2, e0, e1, e2, ix0, ix1, ix2,
                zero2_v, zero1_v, xbar_sh, z_sh,
                ls0, ls1, ls2, ss0, ss1, ss2, zs0, zs1, zs2):
    rows = (rows0, rows1, rows2)
    ebuf = (e0, e1, e2)
    ixbuf = (ix0, ix1, ix2)
    lsem = (ls0, ls1, ls2)
    ssem = (ss0, ss1, ss2)
    zsem = (zs0, zs1, zs2)
    c = lax.axis_index("c")
    s = lax.axis_index("s")
    w = c * NS + s
    base = w * PER_W

    zvec = jnp.zeros((16,), jnp.float32)

    def _zfill2(i, _):
        for v in range(D // 16):
            zero2_v[i, pl.ds(v * 16, 16)] = zvec
        return 0

    lax.fori_loop(0, ZROWS, _zfill2, 0)

    def _zfill1(i, _):
        zero1_v[pl.ds(i * 16, 16)] = zvec
        return 0

    lax.fori_loop(0, ZPT // 16, _zfill1, 0)

    for j in range(RPT // ZROWS):
        pltpu.sync_copy(zero2_v, xbar_sh.at[pl.ds(s * RPT + j * ZROWS, ZROWS)])
    pltpu.sync_copy(zero1_v, z_sh.at[pl.ds(s * ZPT, ZPT)])
    plsc.subcore_barrier()

    def _start_load(ch, b):
        off = base + ch * CHUNK
        pltpu.async_copy(x_hbm.at[pl.ds(off, CHUNK)], rows[b], lsem[b])
        pltpu.async_copy(e_hbm.at[pl.ds(off, CHUNK)], ebuf[b], lsem[b])
        pltpu.async_copy(ix_hbm.at[pl.ds(off, CHUNK)], ixbuf[b], lsem[b])

    def _wait_load(b):
        pltpu.make_async_copy(x_hbm.at[pl.ds(0, CHUNK)], rows[b],
                              lsem[b]).wait()
        pltpu.make_async_copy(e_hbm.at[pl.ds(0, CHUNK)], ebuf[b],
                              lsem[b]).wait()
        pltpu.make_async_copy(ix_hbm.at[pl.ds(0, CHUNK)], ixbuf[b],
                              lsem[b]).wait()

    def _scale(ch, b):
        rv = rows[b]
        ev = ebuf[b]

        def _group(g, _):
            e16 = ev[pl.ds(g * 16, 16)]
            for j in range(16):
                r = g * 16 + j
                splat = jnp.broadcast_to(e16[j], (16,))
                for v in range(D // 16):
                    sl = pl.ds(v * 16, 16)
                    rv[r, sl] = rv[r, sl] * splat
            return 0

        lax.fori_loop(0, CHUNK // 16, _group, 0)

    def _start_scatter(ch, b):
        pltpu.async_copy(rows[b], xbar_sh.at[ixbuf[b]], ssem[b], add=True)
        pltpu.async_copy(ebuf[b], z_sh.at[ixbuf[b]], zsem[b], add=True)

    def _wait_scatter(b):
        pltpu.make_async_copy(rows[b], xbar_sh.at[pl.ds(0, CHUNK)],
                              ssem[b]).wait()
        pltpu.make_async_copy(ebuf[b], z_sh.at[pl.ds(0, CHUNK)],
                              zsem[b]).wait()

    _start_load(0, 0)
    _start_load(1, 1)

    def _steady(p, _):
        for j in range(NB):
            ch = NB * p + j
            b = j
            bp = (j + 2) % NB
            _wait_load(b)
            _scale(ch, b)
            _start_scatter(ch, b)
            if j == 0:
                @pl.when(p > 0)
                def _():
                    _wait_scatter(bp)
                    _start_load(ch + 2, bp)

                @pl.when(p == 0)
                def _():
                    _start_load(ch + 2, bp)
            else:
                _wait_scatter(bp)
                _start_load(ch + 2, bp)
        return 0

    n_steady = (N_CH - 2) // NB
    lax.fori_loop(0, n_steady, _steady, 0)
    for ch in range(n_steady * NB, N_CH):
        b = ch % NB
        _wait_load(b)
        _scale(ch, b)
        _start_scatter(ch, b)
        _wait_scatter((ch + 2) % NB)
    _wait_scatter((N_CH - 1) % NB)
    plsc.subcore_barrier()

    pltpu.sync_copy(xbar_sh.at[pl.ds(s * RPT, RPT)],
                    xbar_hbm.at[c, pl.ds(s * RPT, RPT)])
    pltpu.sync_copy(z_sh.at[pl.ds(s * ZPT, ZPT)],
                    z_hbm.at[c, pl.ds(s * ZPT, ZPT)])


def _combine_body(xbar_ref, z_ref, wf_ref, bf_ref, wh_ref, bh_ref, seg_ref):
    xbar = xbar_ref[0, :S] + xbar_ref[1, :S]
    z = z_ref[0, :S] + z_ref[1, :S]
    inv = jnp.where(z > 0, 1.0 / jnp.where(z > 0, z, 1.0), 0.0)
    ybar = xbar * inv[:, None]
    t = jnp.dot(ybar, wf_ref[...], preferred_element_type=jnp.float32)
    t = t + bf_ref[...]
    o = jnp.dot(t, wh_ref[...], preferred_element_type=jnp.float32)
    seg_ref[...] = o + bh_ref[...]


def _combine(xbar2, z2, Wf, bf, Wh, bh):
    return pl.pallas_call(
        _combine_body,
        out_shape=jax.ShapeDtypeStruct((S, D), jnp.float32),
    )(xbar2, z2, Wf, bf.reshape(1, D), Wh, bh.reshape(1, D))


@functools.partial(
    pl.kernel,
    out_type=jax.ShapeDtypeStruct((N, D), jnp.float32),
    mesh=_mesh,
    scratch_types=[
        pltpu.VMEM((CHUNK, D), jnp.float32),
        pltpu.VMEM((CHUNK, D), jnp.float32),
        pltpu.VMEM((CHUNK, D), jnp.float32),
        pltpu.VMEM((PER_W,), jnp.int32),
        pltpu.SemaphoreType.DMA,
        pltpu.SemaphoreType.DMA,
        pltpu.SemaphoreType.DMA,
        pltpu.SemaphoreType.DMA,
        pltpu.SemaphoreType.DMA,
        pltpu.SemaphoreType.DMA,
    ],
)
def _expand(seg_hbm, ix_hbm, out_hbm, rows0, rows1, rows2, ix_all,
            gs0, gs1, gs2, ts0, ts1, ts2):
    rows = (rows0, rows1, rows2)
    gsem = (gs0, gs1, gs2)
    tsem = (ts0, ts1, ts2)
    c = lax.axis_index("c")
    s = lax.axis_index("s")
    base = (c * NS + s) * PER_W

    pltpu.sync_copy(ix_hbm.at[pl.ds(base, PER_W)], ix_all)

    def _start_gather(ch, b):
        pltpu.async_copy(seg_hbm.at[ix_all.at[pl.ds(ch * CHUNK, CHUNK)]],
                         rows[b], gsem[b])

    def _wait_gather(b):
        pltpu.make_async_copy(seg_hbm.at[pl.ds(0, CHUNK)], rows[b],
                              gsem[b]).wait()

    def _start_store(ch, b):
        pltpu.async_copy(rows[b], out_hbm.at[pl.ds(base + ch * CHUNK, CHUNK)],
                         tsem[b])

    def _wait_store(b):
        pltpu.make_async_copy(rows[b], out_hbm.at[pl.ds(0, CHUNK)],
                              tsem[b]).wait()

    _start_gather(0, 0)
    _start_gather(1, 1)

    def _steady(p, _):
        for j in range(NB):
            ch = NB * p + j
            b = j
            bp = (j + 2) % NB
            _wait_gather(b)
            _start_store(ch, b)
            if j == 0:
                @pl.when(p > 0)
                def _():
                    _wait_store(bp)
                    _start_gather(ch + 2, bp)

                @pl.when(p == 0)
                def _():
                    _start_gather(ch + 2, bp)
            else:
                _wait_store(bp)
                _start_gather(ch + 2, bp)
        return 0

    n_steady = (N_CH - 2) // NB
    lax.fori_loop(0, n_steady, _steady, 0)
    for ch in range(n_steady * NB, N_CH):
        b = ch % NB
        _wait_gather(b)
        _start_store(ch, b)
        _wait_store((ch + 2) % NB)
    _wait_store((N_CH - 1) % NB)


def kernel(x, ix, Wf, bf, Wg, bg, Wh, bh):
    x2 = x.reshape(N, D)
    ixi = ix.reshape(N).astype(jnp.int32)
    e = _gates(x2, Wg, bg).reshape(N)
    xbar2, z2 = _accumulate(x2, e, ixi)
    seg = _combine(xbar2, z2, Wf, bf, Wh, bh)
    out = _expand(seg, ixi)
    return out.reshape(1, N, D)


# speedup vs baseline: 6.2594x; 1.6619x over previous
import functools

import jax
import jax.numpy as jnp
from jax import lax
from jax.experimental import pallas as pl
from jax.experimental.pallas import tpu as pltpu
from jax.experimental.pallas import tpu_sc as plsc

N = 320000
D = 128
S = 10000
SZ = 10240
NC, NS = 2, 16
NW = NC * NS
PER_W = N // NW
CHUNK = 80
N_CH = PER_W // CHUNK
ZROWS = 32
RPT = SZ // NS
ZPT = SZ // NS

_mesh = plsc.VectorSubcoreMesh(core_axis_name="c", subcore_axis_name="s")


def _gates_body(x_ref, wg_ref, bg_ref, e_ref):
    g = jnp.dot(x_ref[...], wg_ref[...], preferred_element_type=jnp.float32)
    e_ref[...] = jnp.exp(g + bg_ref[...])


def _gates(x2, Wg, bg):
    B1 = 4000
    return pl.pallas_call(
        _gates_body,
        grid=(N // B1,),
        in_specs=[
            pl.BlockSpec((B1, D), lambda i: (i, 0)),
            pl.BlockSpec((D, 1), lambda i: (0, 0)),
            pl.BlockSpec((1, 1), lambda i: (0, 0)),
        ],
        out_specs=pl.BlockSpec((B1, 1), lambda i: (i, 0)),
        out_shape=jax.ShapeDtypeStruct((N, 1), jnp.float32),
    )(x2, Wg, bg.reshape(1, 1))


NB = 3


@functools.partial(
    pl.kernel,
    out_type=(
        jax.ShapeDtypeStruct((NC, SZ, D), jnp.float32),
        jax.ShapeDtypeStruct((NC, SZ), jnp.float32),
    ),
    mesh=_mesh,
    scratch_types=[
        pltpu.VMEM((CHUNK, D), jnp.float32),
        pltpu.VMEM((CHUNK, D), jnp.float32),
        pltpu.VMEM((CHUNK, D), jnp.float32),
        pltpu.VMEM((CHUNK,), jnp.float32),
        pltpu.VMEM((CHUNK,), jnp.float32),
        pltpu.VMEM((CHUNK,), jnp.float32),
        pltpu.VMEM((CHUNK,), jnp.int32),
        pltpu.VMEM((CHUNK,), jnp.int32),
        pltpu.VMEM((CHUNK,), jnp.int32),
        pltpu.VMEM((ZROWS, D), jnp.float32),
        pltpu.VMEM((ZPT,), jnp.float32),
        pltpu.VMEM_SHARED((SZ, D), jnp.float32),
        pltpu.VMEM_SHARED((SZ,), jnp.float32),
        pltpu.SemaphoreType.DMA,
        pltpu.SemaphoreType.DMA,
        pltpu.SemaphoreType.DMA,
        pltpu.SemaphoreType.DMA,
        pltpu.SemaphoreType.DMA,
        pltpu.SemaphoreType.DMA,
        pltpu.SemaphoreType.DMA,
        pltpu.SemaphoreType.DMA,
        pltpu.SemaphoreType.DMA,
    ],
)
def _accumulate(x_hbm, e_hbm, ix_hbm, xbar_hbm, z_hbm,
                rows0, rows1, rows2, e0, e1, e2, ix0, ix1, ix2,
                zero2_v, zero1_v, xbar_sh, z_sh,
                ls0, ls1, ls2, ss0, ss1, ss2, zs0, zs1, zs2):
    rows = (rows0, rows1, rows2)
    ebuf = (e0, e1, e2)
    ixbuf = (ix0, ix1, ix2)
    lsem = (ls0, ls1, ls2)
    ssem = (ss0, ss1, ss2)
    zsem = (zs0, zs1, zs2)
    c = lax.axis_index("c")
    s = lax.axis_index("s")
    w = c * NS + s
    base = w * PER_W

    zvec = jnp.zeros((16,), jnp.float32)

    def _zfill2(i, _):
        for v in range(D // 16):
            zero2_v[i, pl.ds(v * 16, 16)] = zvec
        return 0

    lax.fori_loop(0, ZROWS, _zfill2, 0)

    def _zfill1(i, _):
        zero1_v[pl.ds(i * 16, 16)] = zvec
        return 0

    lax.fori_loop(0, ZPT // 16, _zfill1, 0)

    for j in range(RPT // ZROWS):
        pltpu.sync_copy(zero2_v, xbar_sh.at[pl.ds(s * RPT + j * ZROWS, ZROWS)])
    pltpu.sync_copy(zero1_v, z_sh.at[pl.ds(s * ZPT, ZPT)])
    plsc.subcore_barrier()

    def _start_load(ch, b):
        off = base + ch * CHUNK
        pltpu.async_copy(x_hbm.at[pl.ds(off, CHUNK)], rows[b], lsem[b])
        pltpu.async_copy(e_hbm.at[pl.ds(off, CHUNK)], ebuf[b], lsem[b])
        pltpu.async_copy(ix_hbm.at[pl.ds(off, CHUNK)], ixbuf[b], lsem[b])

    def _wait_load(b):
        pltpu.make_async_copy(x_hbm.at[pl.ds(0, CHUNK)], rows[b],
                              lsem[b]).wait()
        pltpu.make_async_copy(e_hbm.at[pl.ds(0, CHUNK)], ebuf[b],
                              lsem[b]).wait()
        pltpu.make_async_copy(ix_hbm.at[pl.ds(0, CHUNK)], ixbuf[b],
                              lsem[b]).wait()

    def _scale(ch, b):
        rv = rows[b]
        ev = ebuf[b]

        def _group(g, _):
            e16 = ev[pl.ds(g * 16, 16)]
            for j in range(16):
                r = g * 16 + j
                splat = jnp.broadcast_to(e16[j], (16,))
                for v in range(D // 16):
                    sl = pl.ds(v * 16, 16)
                    rv[r, sl] = rv[r, sl] * splat
            return 0

        lax.fori_loop(0, CHUNK // 16, _group, 0)

    def _start_scatter(ch, b):
        pltpu.async_copy(rows[b], xbar_sh.at[ixbuf[b]], ssem[b], add=True)
        pltpu.async_copy(ebuf[b], z_sh.at[ixbuf[b]], zsem[b], add=True)

    def _wait_scatter(b):
        pltpu.make_async_copy(rows[b], xbar_sh.at[pl.ds(0, CHUNK)],
                              ssem[b]).wait()
        pltpu.make_async_copy(ebuf[b], z_sh.at[pl.ds(0, CHUNK)],
                              zsem[b]).wait()

    _start_load(0, 0)
    _start_load(1, 1)

    def _steady(p, _):
        for j in range(NB):
            ch = NB * p + j
            b = j
            bp = (j + 2) % NB
            _wait_load(b)
            _scale(ch, b)
            _start_scatter(ch, b)
            if j == 0:
                @pl.when(p > 0)
                def _():
                    _wait_scatter(bp)
                    _start_load(ch + 2, bp)

                @pl.when(p == 0)
                def _():
                    _start_load(ch + 2, bp)
            else:
                _wait_scatter(bp)
                _start_load(ch + 2, bp)
        return 0

    n_steady = (N_CH - 2) // NB
    lax.fori_loop(0, n_steady, _steady, 0)
    for ch in range(n_steady * NB, N_CH):
        b = ch % NB
        _wait_load(b)
        _scale(ch, b)
        _start_scatter(ch, b)
        _wait_scatter((ch + 2) % NB)
    _wait_scatter((N_CH - 1) % NB)
    plsc.subcore_barrier()

    pltpu.sync_copy(xbar_sh.at[pl.ds(s * RPT, RPT)],
                    xbar_hbm.at[c, pl.ds(s * RPT, RPT)])
    pltpu.sync_copy(z_sh.at[pl.ds(s * ZPT, ZPT)],
                    z_hbm.at[c, pl.ds(s * ZPT, ZPT)])


def _combine_body(xbar_ref, z_ref, wf_ref, bf_ref, wh_ref, bh_ref, seg_ref):
    xbar = xbar_ref[0] + xbar_ref[1]
    z = z_ref[0] + z_ref[1]
    inv = jnp.where(z > 0, 1.0 / jnp.where(z > 0, z, 1.0), 0.0)
    ybar = xbar * inv[:, None]
    t = jnp.dot(ybar, wf_ref[...], preferred_element_type=jnp.float32)
    t = t + bf_ref[...]
    o = jnp.dot(t, wh_ref[...], preferred_element_type=jnp.float32)
    seg_ref[...] = o + bh_ref[...]


def _combine(xbar2, z2, Wf, bf, Wh, bh):
    return pl.pallas_call(
        _combine_body,
        out_shape=jax.ShapeDtypeStruct((SZ, D), jnp.float32),
    )(xbar2, z2, Wf, bf.reshape(1, D), Wh, bh.reshape(1, D))


@functools.partial(
    pl.kernel,
    out_type=jax.ShapeDtypeStruct((N, D), jnp.float32),
    mesh=_mesh,
    scratch_types=[
        pltpu.VMEM((CHUNK, D), jnp.float32),
        pltpu.VMEM((CHUNK, D), jnp.float32),
        pltpu.VMEM((CHUNK, D), jnp.float32),
        pltpu.VMEM((PER_W,), jnp.int32),
        pltpu.VMEM_SHARED((SZ, D), jnp.float32),
        pltpu.SemaphoreType.DMA,
        pltpu.SemaphoreType.DMA,
        pltpu.SemaphoreType.DMA,
        pltpu.SemaphoreType.DMA,
        pltpu.SemaphoreType.DMA,
        pltpu.SemaphoreType.DMA,
    ],
)
def _expand(seg_hbm, ix_hbm, out_hbm, rows0, rows1, rows2, ix_all, seg_sh,
            gs0, gs1, gs2, ts0, ts1, ts2):
    rows = (rows0, rows1, rows2)
    gsem = (gs0, gs1, gs2)
    tsem = (ts0, ts1, ts2)
    c = lax.axis_index("c")
    s = lax.axis_index("s")
    base = (c * NS + s) * PER_W

    pltpu.sync_copy(seg_hbm.at[pl.ds(s * ZPT, ZPT)],
                    seg_sh.at[pl.ds(s * ZPT, ZPT)])
    pltpu.sync_copy(ix_hbm.at[pl.ds(base, PER_W)], ix_all)
    plsc.subcore_barrier()

    def _start_gather(ch, b):
        pltpu.async_copy(seg_sh.at[ix_all.at[pl.ds(ch * CHUNK, CHUNK)]],
                         rows[b], gsem[b])

    def _wait_gather(b):
        pltpu.make_async_copy(seg_hbm.at[pl.ds(0, CHUNK)], rows[b],
                              gsem[b]).wait()

    def _start_store(ch, b):
        pltpu.async_copy(rows[b], out_hbm.at[pl.ds(base + ch * CHUNK, CHUNK)],
                         tsem[b])

    def _wait_store(b):
        pltpu.make_async_copy(rows[b], out_hbm.at[pl.ds(0, CHUNK)],
                              tsem[b]).wait()

    _start_gather(0, 0)
    _start_gather(1, 1)

    def _steady(p, _):
        for j in range(NB):
            ch = NB * p + j
            b = j
            bp = (j + 2) % NB
            _wait_gather(b)
            _start_store(ch, b)
            if j == 0:
                @pl.when(p > 0)
                def _():
                    _wait_store(bp)
                    _start_gather(ch + 2, bp)

                @pl.when(p == 0)
                def _():
                    _start_gather(ch + 2, bp)
            else:
                _wait_store(bp)
                _start_gather(ch + 2, bp)
        return 0

    n_steady = (N_CH - 2) // NB
    lax.fori_loop(0, n_steady, _steady, 0)
    for ch in range(n_steady * NB, N_CH):
        b = ch % NB
        _wait_gather(b)
        _start_store(ch, b)
        _wait_store((ch + 2) % NB)
    _wait_store((N_CH - 1) % NB)


def kernel(x, ix, Wf, bf, Wg, bg, Wh, bh):
    x2 = x.reshape(N, D)
    ixi = ix.reshape(N).astype(jnp.int32)
    e = _gates(x2, Wg, bg).reshape(N)
    xbar2, z2 = _accumulate(x2, e, ixi)
    seg = _combine(xbar2, z2, Wf, bf, Wh, bh)
    out = _expand(seg, ixi)
    return out.reshape(1, N, D)


# speedup vs baseline: 9.8132x; 1.5678x over previous
import functools

import jax
import jax.numpy as jnp
import numpy as np
from jax import lax
from jax.experimental import pallas as pl
from jax.experimental.pallas import tpu as pltpu
from jax.experimental.pallas import tpu_sc as plsc

N = 320000
D = 128
S = 10000
SZ = 10240
NC, NS = 2, 16
NW = NC * NS
PER_W = N // NW
CHUNK = 80
N_CH = PER_W // CHUNK
ZROWS = 32
RPT = SZ // NS
ZPT = SZ // NS

_mesh = plsc.VectorSubcoreMesh(core_axis_name="c", subcore_axis_name="s")


_DN = lax.GatherDimensionNumbers(
    offset_dims=(), collapsed_slice_dims=(0,), start_index_map=(0,))


def _hsum_splat(v, lane):
    for sh in (8, 4, 2, 1):
        idx = (lane + sh) & 15
        g = lax.gather(v, idx[:, None], _DN, slice_sizes=(1,),
                       mode=lax.GatherScatterMode.PROMISE_IN_BOUNDS)
        v = v + g
    return v

NB = 3


@functools.partial(
    pl.kernel,
    out_type=(
        jax.ShapeDtypeStruct((NC, SZ, D), jnp.float32),
        jax.ShapeDtypeStruct((NC, SZ), jnp.float32),
    ),
    mesh=_mesh,
    scratch_types=[
        pltpu.VMEM((CHUNK, D), jnp.float32),
        pltpu.VMEM((CHUNK, D), jnp.float32),
        pltpu.VMEM((CHUNK, D), jnp.float32),
        pltpu.VMEM((CHUNK,), jnp.float32),
        pltpu.VMEM((CHUNK,), jnp.float32),
        pltpu.VMEM((CHUNK,), jnp.float32),
        pltpu.VMEM((CHUNK,), jnp.int32),
        pltpu.VMEM((CHUNK,), jnp.int32),
        pltpu.VMEM((CHUNK,), jnp.int32),
        pltpu.VMEM((D,), jnp.float32),
        pltpu.VMEM((ZROWS, D), jnp.float32),
        pltpu.VMEM((ZPT,), jnp.float32),
        pltpu.VMEM_SHARED((SZ, D), jnp.float32),
        pltpu.VMEM_SHARED((SZ,), jnp.float32),
        pltpu.SemaphoreType.DMA,
        pltpu.SemaphoreType.DMA,
        pltpu.SemaphoreType.DMA,
        pltpu.SemaphoreType.DMA,
        pltpu.SemaphoreType.DMA,
        pltpu.SemaphoreType.DMA,
        pltpu.SemaphoreType.DMA,
        pltpu.SemaphoreType.DMA,
        pltpu.SemaphoreType.DMA,
    ],
)
def _accumulate(x_hbm, ix_hbm, wg_hbm, xbar_hbm, z_hbm,
                rows0, rows1, rows2, e0, e1, e2, ix0, ix1, ix2,
                wg_v, zero2_v, zero1_v, xbar_sh, z_sh,
                ls0, ls1, ls2, ss0, ss1, ss2, zs0, zs1, zs2):
    rows = (rows0, rows1, rows2)
    ebuf = (e0, e1, e2)
    ixbuf = (ix0, ix1, ix2)
    lsem = (ls0, ls1, ls2)
    ssem = (ss0, ss1, ss2)
    zsem = (zs0, zs1, zs2)
    c = lax.axis_index("c")
    s = lax.axis_index("s")
    w = c * NS + s
    base = w * PER_W

    zvec = jnp.zeros((16,), jnp.float32)

    def _zfill2(i, _):
        for v in range(D // 16):
            zero2_v[i, pl.ds(v * 16, 16)] = zvec
        return 0

    lax.fori_loop(0, ZROWS, _zfill2, 0)

    def _zfill1(i, _):
        zero1_v[pl.ds(i * 16, 16)] = zvec
        return 0

    lax.fori_loop(0, ZPT // 16, _zfill1, 0)

    for j in range(RPT // ZROWS):
        pltpu.sync_copy(zero2_v, xbar_sh.at[pl.ds(s * RPT + j * ZROWS, ZROWS)])
    pltpu.sync_copy(zero1_v, z_sh.at[pl.ds(s * ZPT, ZPT)])
    pltpu.sync_copy(wg_hbm, wg_v)
    plsc.subcore_barrier()

    wg = [wg_v[pl.ds(k * 16, 16)] for k in range(D // 16)]
    lane = lax.broadcasted_iota(jnp.int32, (16,), 0)

    def _start_load(ch, b):
        off = base + ch * CHUNK
        pltpu.async_copy(x_hbm.at[pl.ds(off, CHUNK)], rows[b], lsem[b])
        pltpu.async_copy(ix_hbm.at[pl.ds(off, CHUNK)], ixbuf[b], lsem[b])

    def _wait_load(b):
        pltpu.make_async_copy(x_hbm.at[pl.ds(0, CHUNK)], rows[b],
                              lsem[b]).wait()
        pltpu.make_async_copy(ix_hbm.at[pl.ds(0, CHUNK)], ixbuf[b],
                              lsem[b]).wait()

    def _scale(ch, b):
        rv = rows[b]
        ev = ebuf[b]

        def _group(g, _):
            acc = jnp.zeros((16,), jnp.float32)
            for j in range(16):
                r = g * 16 + j
                v = [rv[r, pl.ds(k * 16, 16)] for k in range(D // 16)]
                ps = v[0] * wg[0]
                for k in range(1, D // 16):
                    ps = ps + v[k] * wg[k]
                e16 = jnp.exp(_hsum_splat(ps, lane))
                acc = jnp.where(lane == j, e16, acc)
                for k in range(D // 16):
                    rv[r, pl.ds(k * 16, 16)] = v[k] * e16
            ev[pl.ds(g * 16, 16)] = acc
            return 0

        lax.fori_loop(0, CHUNK // 16, _group, 0)

    def _start_scatter(ch, b):
        pltpu.async_copy(rows[b], xbar_sh.at[ixbuf[b]], ssem[b], add=True)
        pltpu.async_copy(ebuf[b], z_sh.at[ixbuf[b]], zsem[b], add=True)

    def _wait_scatter(b):
        pltpu.make_async_copy(rows[b], xbar_sh.at[pl.ds(0, CHUNK)],
                              ssem[b]).wait()
        pltpu.make_async_copy(ebuf[b], z_sh.at[pl.ds(0, CHUNK)],
                              zsem[b]).wait()

    _start_load(0, 0)
    _start_load(1, 1)

    def _steady(p, _):
        for j in range(NB):
            ch = NB * p + j
            b = j
            bp = (j + 2) % NB
            _wait_load(b)
            _scale(ch, b)
            _start_scatter(ch, b)
            if j == 0:
                @pl.when(p > 0)
                def _():
                    _wait_scatter(bp)
                    _start_load(ch + 2, bp)

                @pl.when(p == 0)
                def _():
                    _start_load(ch + 2, bp)
            else:
                _wait_scatter(bp)
                _start_load(ch + 2, bp)
        return 0

    n_steady = (N_CH - 2) // NB
    lax.fori_loop(0, n_steady, _steady, 0)
    for ch in range(n_steady * NB, N_CH):
        b = ch % NB
        _wait_load(b)
        _scale(ch, b)
        _start_scatter(ch, b)
        _wait_scatter((ch + 2) % NB)
    _wait_scatter((N_CH - 1) % NB)
    plsc.subcore_barrier()

    pltpu.sync_copy(xbar_sh.at[pl.ds(s * RPT, RPT)],
                    xbar_hbm.at[c, pl.ds(s * RPT, RPT)])
    pltpu.sync_copy(z_sh.at[pl.ds(s * ZPT, ZPT)],
                    z_hbm.at[c, pl.ds(s * ZPT, ZPT)])


def _combine_body(xbar_ref, z_ref, wf_ref, bf_ref, wh_ref, bh_ref, seg_ref):
    xbar = xbar_ref[0] + xbar_ref[1]
    z = z_ref[0] + z_ref[1]
    inv = jnp.where(z > 0, 1.0 / jnp.where(z > 0, z, 1.0), 0.0)
    ybar = xbar * inv[:, None]
    t = jnp.dot(ybar, wf_ref[...], preferred_element_type=jnp.float32)
    t = t + bf_ref[...]
    o = jnp.dot(t, wh_ref[...], preferred_element_type=jnp.float32)
    seg_ref[...] = o + bh_ref[...]


def _combine(xbar2, z2, Wf, bf, Wh, bh):
    return pl.pallas_call(
        _combine_body,
        out_shape=jax.ShapeDtypeStruct((SZ, D), jnp.float32),
    )(xbar2, z2, Wf, bf.reshape(1, D), Wh, bh.reshape(1, D))


@functools.partial(
    pl.kernel,
    out_type=jax.ShapeDtypeStruct((N, D), jnp.float32),
    mesh=_mesh,
    scratch_types=[
        pltpu.VMEM((CHUNK, D), jnp.float32),
        pltpu.VMEM((CHUNK, D), jnp.float32),
        pltpu.VMEM((CHUNK, D), jnp.float32),
        pltpu.VMEM((PER_W,), jnp.int32),
        pltpu.VMEM_SHARED((SZ, D), jnp.float32),
        pltpu.SemaphoreType.DMA,
        pltpu.SemaphoreType.DMA,
        pltpu.SemaphoreType.DMA,
        pltpu.SemaphoreType.DMA,
        pltpu.SemaphoreType.DMA,
        pltpu.SemaphoreType.DMA,
    ],
)
def _expand(seg_hbm, ix_hbm, out_hbm, rows0, rows1, rows2, ix_all, seg_sh,
            gs0, gs1, gs2, ts0, ts1, ts2):
    rows = (rows0, rows1, rows2)
    gsem = (gs0, gs1, gs2)
    tsem = (ts0, ts1, ts2)
    c = lax.axis_index("c")
    s = lax.axis_index("s")
    base = (c * NS + s) * PER_W

    pltpu.sync_copy(seg_hbm.at[pl.ds(s * ZPT, ZPT)],
                    seg_sh.at[pl.ds(s * ZPT, ZPT)])
    pltpu.sync_copy(ix_hbm.at[pl.ds(base, PER_W)], ix_all)
    plsc.subcore_barrier()

    def _start_gather(ch, b):
        pltpu.async_copy(seg_sh.at[ix_all.at[pl.ds(ch * CHUNK, CHUNK)]],
                         rows[b], gsem[b])

    def _wait_gather(b):
        pltpu.make_async_copy(seg_hbm.at[pl.ds(0, CHUNK)], rows[b],
                              gsem[b]).wait()

    def _start_store(ch, b):
        pltpu.async_copy(rows[b], out_hbm.at[pl.ds(base + ch * CHUNK, CHUNK)],
                         tsem[b])

    def _wait_store(b):
        pltpu.make_async_copy(rows[b], out_hbm.at[pl.ds(0, CHUNK)],
                              tsem[b]).wait()

    _start_gather(0, 0)
    _start_gather(1, 1)

    def _steady(p, _):
        for j in range(NB):
            ch = NB * p + j
            b = j
            bp = (j + 2) % NB
            _wait_gather(b)
            _start_store(ch, b)
            if j == 0:
                @pl.when(p > 0)
                def _():
                    _wait_store(bp)
                    _start_gather(ch + 2, bp)

                @pl.when(p == 0)
                def _():
                    _start_gather(ch + 2, bp)
            else:
                _wait_store(bp)
                _start_gather(ch + 2, bp)
        return 0

    n_steady = (N_CH - 2) // NB
    lax.fori_loop(0, n_steady, _steady, 0)
    for ch in range(n_steady * NB, N_CH):
        b = ch % NB
        _wait_gather(b)
        _start_store(ch, b)
        _wait_store((ch + 2) % NB)
    _wait_store((N_CH - 1) % NB)


def kernel(x, ix, Wf, bf, Wg, bg, Wh, bh):
    x2 = x.reshape(N, D)
    ixi = ix.reshape(N).astype(jnp.int32)
    xbar2, z2 = _accumulate(x2, ixi, Wg.reshape(D))
    seg = _combine(xbar2, z2, Wf, bf, Wh, bh)
    out = _expand(seg, ixi)
    return out.reshape(1, N, D)
